# baseline (speedup 1.0000x reference)
import jax
import jax.numpy as jnp
from jax import lax
from jax.experimental import pallas as pl
from jax.experimental.pallas import tpu as pltpu
from jax.experimental.pallas import tpu_sc as plsc

NUM_FIELDS = 26
VOCAB = 100000
EMBED_DIM = 32
BATCH = 16384

NUM_CORES = 2
NUM_SUBCORES = 16
NUM_WORKERS = NUM_CORES * NUM_SUBCORES
BPW = BATCH // NUM_WORKERS
CHUNK = 128
NCHUNK = BPW // CHUNK
GROUPS = BPW // 16
GPC = CHUNK // 16


def _body(inst_hbm, w_hbm, out_hbm, idx_v, gidx_v, rbuf_v,
          gsem0, gsem1, wsem0, wsem1):
    gsems = (gsem0, gsem1)
    wsems = (wsem0, wsem1)
    wid = lax.axis_index("s") * NUM_CORES + lax.axis_index("c")
    base = wid * BPW
    pltpu.sync_copy(inst_hbm.at[pl.ds(base, BPW)], idx_v)

    def compute_gidx(f, s):
        off = f * VOCAB
        for g in range(GROUPS):
            gidx_v[s, g // GPC, pl.ds((g % GPC) * 16, 16)] = (
                idx_v[pl.ds(g * 16, 16)] + off
            )

    def issue_gathers(s):
        for c in range(NCHUNK):
            pltpu.async_copy(
                w_hbm.at[gidx_v.at[s, c]],
                rbuf_v.at[s, pl.ds(c * CHUNK, CHUNK)],
                gsems[s],
            )

    def wait_gathers(s):
        for c in range(NCHUNK):
            pltpu.make_async_copy(
                w_hbm.at[gidx_v.at[s, c]],
                rbuf_v.at[s, pl.ds(c * CHUNK, CHUNK)],
                gsems[s],
            ).wait()

    def issue_wb(f, s):
        pltpu.async_copy(rbuf_v.at[s], out_hbm.at[pl.ds(base, BPW), f],
                         wsems[s])

    def wait_wb(s):
        pltpu.make_async_copy(rbuf_v.at[s], out_hbm.at[pl.ds(base, BPW), 0],
                              wsems[s]).wait()

    compute_gidx(0, 0)
    issue_gathers(0)

    def step(i, carry):
        for b in range(2):
            fld = 2 * i + b
            s = b
            wait_gathers(s)
            issue_wb(fld, s)

            @pl.when(fld + 1 < NUM_FIELDS)
            def _():
                compute_gidx(fld + 1, 1 - s)

                @pl.when(fld >= 1)
                def _():
                    wait_wb(1 - s)

                issue_gathers(1 - s)
        return carry

    lax.fori_loop(0, NUM_FIELDS // 2, step, 0)
    wait_wb(0)
    wait_wb(1)


def kernel(instance, W):
    w_flat = W.reshape(NUM_FIELDS * VOCAB, EMBED_DIM)
    idx = instance.astype(jnp.int32)
    mesh = plsc.VectorSubcoreMesh(core_axis_name="c", subcore_axis_name="s")
    out = pl.kernel(
        _body,
        out_type=jax.ShapeDtypeStruct((BATCH, NUM_FIELDS, EMBED_DIM), jnp.float32),
        mesh=mesh,
        scratch_types=[
            pltpu.VMEM((BPW,), jnp.int32),
            pltpu.VMEM((2, NCHUNK, CHUNK), jnp.int32),
            pltpu.VMEM((2, BPW, EMBED_DIM), jnp.float32),
            pltpu.SemaphoreType.DMA,
            pltpu.SemaphoreType.DMA,
            pltpu.SemaphoreType.DMA,
            pltpu.SemaphoreType.DMA,
        ],
        compiler_params=pltpu.CompilerParams(use_tc_tiling_on_sc=False),
    )(idx, w_flat)
    return out.reshape(BATCH, NUM_FIELDS * EMBED_DIM)


# speedup vs baseline: 1.1314x; 1.1314x over previous
import jax
import jax.numpy as jnp
from jax import lax
from jax.experimental import pallas as pl
from jax.experimental.pallas import tpu as pltpu
from jax.experimental.pallas import tpu_sc as plsc

NUM_FIELDS = 26
VOCAB = 100000
EMBED_DIM = 32
BATCH = 16384

NUM_CORES = 2
NUM_SUBCORES = 16
NUM_WORKERS = NUM_CORES * NUM_SUBCORES
BPW = BATCH // NUM_WORKERS
CHUNK = 128
NCHUNK = BPW // CHUNK
NBUF = 6
LOOKAHEAD = 4


def _body(inst_hbm, w_hbm, out_hbm, gidx_v, buf_v, *sems):
    gsems = sems[:NBUF]
    wsems = sems[NBUF:]
    wid = lax.axis_index("s") * NUM_CORES + lax.axis_index("c")
    base = wid * BPW
    for c in range(NCHUNK):
        pltpu.sync_copy(inst_hbm.at[pl.ds(base + c * CHUNK, CHUNK)],
                        gidx_v.at[c])

    def gather_pairs(f, s):
        return [(
            w_hbm.at[f].at[gidx_v.at[c]],
            buf_v.at[s, pl.ds(c * CHUNK, CHUNK), :],
        ) for c in range(NCHUNK)]

    def wb_pair(f, s):
        return (
            buf_v.at[s],
            out_hbm.at[pl.ds(base, BPW), pl.ds(f * EMBED_DIM, EMBED_DIM)],
        )

    def issue_gathers(f, s):
        for src, dst in gather_pairs(f, s):
            pltpu.async_copy(src, dst, gsems[s])

    def wait_gathers(f, s):
        for src, dst in gather_pairs(f, s):
            pltpu.make_async_copy(src, dst, gsems[s]).wait()

    def issue_wb(f, s):
        src, dst = wb_pair(f, s)
        pltpu.async_copy(src, dst, wsems[s])

    def wait_wb(f, s):
        src, dst = wb_pair(f, s)
        pltpu.make_async_copy(src, dst, wsems[s]).wait()

    for f in range(min(LOOKAHEAD, NUM_FIELDS)):
        issue_gathers(f, f % NBUF)
    for f in range(NUM_FIELDS):
        s = f % NBUF
        nf = f + LOOKAHEAD
        if nf < NUM_FIELDS:
            ps = nf % NBUF
            if nf - NBUF >= 0:
                wait_wb(nf - NBUF, ps)
            issue_gathers(nf, ps)
        wait_gathers(f, s)
        issue_wb(f, s)
    for f in range(max(0, NUM_FIELDS - NBUF), NUM_FIELDS):
        wait_wb(f, f % NBUF)


def kernel(instance, W):
    idx = instance.astype(jnp.int32)
    mesh = plsc.VectorSubcoreMesh(core_axis_name="c", subcore_axis_name="s")
    out = pl.kernel(
        _body,
        out_type=jax.ShapeDtypeStruct((BATCH, NUM_FIELDS * EMBED_DIM), jnp.float32),
        mesh=mesh,
        scratch_types=[
            pltpu.VMEM((NCHUNK, CHUNK), jnp.int32),
            pltpu.VMEM((NBUF, BPW, EMBED_DIM), jnp.float32),
        ] + [pltpu.SemaphoreType.DMA] * (2 * NBUF),
        compiler_params=pltpu.CompilerParams(use_tc_tiling_on_sc=False),
    )(idx, W)
    return out
